# baseline (speedup 1.0000x reference)
import jax
import jax.numpy as jnp
from jax.experimental import pallas as pl
from jax.experimental.pallas import tpu as pltpu

BM = 200


def _gcn_body(a0_ref, a1_ref, x_ref, deg_ref, wt_ref, b_ref, out_ref):
    xb = x_ref[...].astype(jnp.bfloat16)
    inv = 1.0 / deg_ref[...]
    for h, a_ref in enumerate((a0_ref, a1_ref)):
        acc = jnp.dot(a_ref[...].astype(jnp.bfloat16), xb,
                      preferred_element_type=jnp.float32)
        xr = x_ref[pl.ds(h * (x_ref.shape[0] // 2)
                         + pl.program_id(0) * BM, BM), :]
        pool = inv[h] * (acc + xr) + xr
        out = jnp.dot(pool, wt_ref[...], preferred_element_type=jnp.float32)
        out_ref[h, ...] = jnp.maximum(out + b_ref[...], 0.0)


@jax.jit
def kernel(input_tensor, adjacency_matrix, node_degree, W, b):
    n, d_in = input_tensor.shape
    d_out = W.shape[0]
    half = n // 2
    grid = half // BM
    wt = W.T
    b2 = b.reshape(1, d_out)
    deg2 = node_degree.reshape(2, half, 1)

    out = pl.pallas_call(
        _gcn_body,
        grid=(grid,),
        in_specs=[
            pl.BlockSpec((BM, n), lambda i: (i, 0)),
            pl.BlockSpec((BM, n), lambda i, g=grid: (g + i, 0)),
            pl.BlockSpec((n, d_in), lambda i: (0, 0)),
            pl.BlockSpec((2, BM, 1), lambda i: (0, i, 0)),
            pl.BlockSpec((d_in, d_out), lambda i: (0, 0)),
            pl.BlockSpec((1, d_out), lambda i: (0, 0)),
        ],
        out_specs=pl.BlockSpec((2, BM, d_out), lambda i: (0, i, 0)),
        out_shape=jax.ShapeDtypeStruct((2, half, d_out), jnp.float32),
        compiler_params=pltpu.CompilerParams(
            dimension_semantics=("parallel",)),
    )(adjacency_matrix, adjacency_matrix, input_tensor, deg2, wt, b2)
    return out.reshape(n, d_out)


# speedup vs baseline: 1.0045x; 1.0045x over previous
import functools

import jax
import jax.numpy as jnp
from jax.experimental import pallas as pl
from jax.experimental.pallas import tpu as pltpu

BM = 256


def _gcn_body(n, a0_ref, a1_ref, x_ref, deg_ref, wt_ref, b_ref, out_ref):
    i = pl.program_id(0)
    xb = x_ref[pl.ds(0, n), :].astype(jnp.bfloat16)
    acc0 = jnp.dot(a0_ref[...].astype(jnp.bfloat16), xb,
                   preferred_element_type=jnp.float32)
    acc1 = jnp.dot(a1_ref[...].astype(jnp.bfloat16), xb,
                   preferred_element_type=jnp.float32)
    acc = jnp.concatenate([acc0, acc1], axis=0)
    xr = x_ref[pl.ds(i * (2 * BM), 2 * BM), :]
    inv = 1.0 / deg_ref[...]
    pool = inv * (acc + xr) + xr
    out = jnp.dot(pool, wt_ref[...], preferred_element_type=jnp.float32)
    out_ref[...] = jnp.maximum(out + b_ref[...], 0.0)


@jax.jit
def kernel(input_tensor, adjacency_matrix, node_degree, W, b):
    n, d_in = input_tensor.shape
    d_out = W.shape[0]
    wt = W.T
    b2 = b.reshape(1, d_out)
    grid = pl.cdiv(n, 2 * BM)
    padn = grid * 2 * BM
    xp = jnp.pad(input_tensor, ((0, padn - n), (0, 0)))

    return pl.pallas_call(
        functools.partial(_gcn_body, n),
        grid=(grid,),
        in_specs=[
            pl.BlockSpec((BM, n), lambda i: (2 * i, 0)),
            pl.BlockSpec((BM, n), lambda i: (2 * i + 1, 0)),
            pl.BlockSpec((padn, d_in), lambda i: (0, 0)),
            pl.BlockSpec((2 * BM, 1), lambda i: (i, 0)),
            pl.BlockSpec((d_in, d_out), lambda i: (0, 0)),
            pl.BlockSpec((1, d_out), lambda i: (0, 0)),
        ],
        out_specs=pl.BlockSpec((2 * BM, d_out), lambda i: (i, 0)),
        out_shape=jax.ShapeDtypeStruct((n, d_out), jnp.float32),
        compiler_params=pltpu.CompilerParams(
            dimension_semantics=("parallel",)),
    )(adjacency_matrix, adjacency_matrix, xp, node_degree, wt, b2)


# speedup vs baseline: 1.0354x; 1.0309x over previous
import jax
import jax.numpy as jnp
from jax.experimental import pallas as pl
from jax.experimental.pallas import tpu as pltpu

BM = 200


def _gcn_body(a0_ref, a1_ref, x_ref, deg_ref, wt_ref, b_ref, out_ref):
    i = pl.program_id(0)
    xb = x_ref[...].astype(jnp.bfloat16)
    acc0 = jnp.dot(a0_ref[...].astype(jnp.bfloat16), xb,
                   preferred_element_type=jnp.float32)
    acc1 = jnp.dot(a1_ref[...].astype(jnp.bfloat16), xb,
                   preferred_element_type=jnp.float32)
    acc = jnp.concatenate([acc0, acc1], axis=0)
    xr = x_ref[pl.ds(i * (2 * BM), 2 * BM), :]
    inv = 1.0 / deg_ref[...]
    pool = inv * (acc + xr) + xr
    out = jnp.dot(pool, wt_ref[...], preferred_element_type=jnp.float32)
    out_ref[...] = jnp.maximum(out + b_ref[...], 0.0)


@jax.jit
def kernel(input_tensor, adjacency_matrix, node_degree, W, b):
    n, d_in = input_tensor.shape
    d_out = W.shape[0]
    wt = W.T
    b2 = b.reshape(1, d_out)

    return pl.pallas_call(
        _gcn_body,
        grid=(n // (2 * BM),),
        in_specs=[
            pl.BlockSpec((BM, n), lambda i: (2 * i, 0)),
            pl.BlockSpec((BM, n), lambda i: (2 * i + 1, 0)),
            pl.BlockSpec((n, d_in), lambda i: (0, 0)),
            pl.BlockSpec((2 * BM, 1), lambda i: (i, 0)),
            pl.BlockSpec((d_in, d_out), lambda i: (0, 0)),
            pl.BlockSpec((1, d_out), lambda i: (0, 0)),
        ],
        out_specs=pl.BlockSpec((2 * BM, d_out), lambda i: (i, 0)),
        out_shape=jax.ShapeDtypeStruct((n, d_out), jnp.float32),
        compiler_params=pltpu.CompilerParams(
            dimension_semantics=("parallel",)),
    )(adjacency_matrix, adjacency_matrix, input_tensor, node_degree, wt, b2)
